# Initial kernel scaffold; baseline (speedup 1.0000x reference)
import jax
import jax.numpy as jnp
from jax.experimental import pallas as pl


def kernel(x, edge_index, p0, p1, gcn0_h, gcn1_h, w_ih0, w_hh0, b_ih0, b_hh0, w_ih1, w_hh1, b_ih1, b_hh1):
    raise NotImplementedError("write your pallas kernel here")


# speedup vs baseline: 11.3562x; 11.3562x over previous
import functools

import jax
import jax.numpy as jnp
from jax import lax
from jax.experimental import pallas as pl
from jax.experimental.pallas import tpu as pltpu
from jax.experimental.pallas import tpu_sc as plsc

_N = 10000
_E = 320000
_D = 128
_DH = 32
_K = 8
_H0 = _D * _DH + _DH
_I0 = _K * _D
_H1 = _DH * _DH + _DH
_I1 = _K * _DH

_NC = 2
_NS = 16
_NTILE = _NC * _NS
_EPT = _E // _NTILE
_CH = 128
_NFULL = _EPT // _CH
_TAIL = _EPT - _NFULL * _CH
_NPAD = 10240
_RPT = _NPAD // _NS

_HI = jax.lax.Precision.HIGHEST
_DEF = jax.lax.Precision.DEFAULT


def _summarize_body(x_ref, p_ref, z_ref, y_ref):
    p = p_ref[...]
    pn = p / (jnp.sqrt(jnp.sum(p * p)) + 1e-8)
    xb = x_ref[...].astype(jnp.bfloat16).astype(jnp.float32)
    pb = pn.astype(jnp.bfloat16).astype(jnp.float32)
    y_ref[...] = jnp.sum(xb * pb, axis=1, keepdims=True)
    n = x_ref.shape[0]
    ids = lax.broadcasted_iota(jnp.int32, (n, 1), 0)
    kiota = lax.broadcasted_iota(jnp.int32, z_ref.shape, 0)

    def body(k, _):
        yv = y_ref[...]
        m = jnp.max(yv)
        idx = jnp.min(jnp.where(yv == m, ids, jnp.int32(n)))
        onehot = jnp.where(ids == idx, 1.0, 0.0)
        row = lax.dot_general(onehot, x_ref[...], (((0,), (0,)), ((), ())),
                              precision=_HI)
        t = jnp.tanh(jnp.full((1, 1), m, jnp.float32))
        z_ref[...] = jnp.where(kiota == k, row * t, z_ref[...])
        y_ref[...] = jnp.where(ids == idx, -jnp.inf, yv)
        return 0

    lax.fori_loop(0, _K, body, 0)


def _summarize(x, p):
    n, d = x.shape
    return pl.pallas_call(
        _summarize_body,
        out_shape=jax.ShapeDtypeStruct((_K, d), jnp.float32),
        scratch_shapes=[pltpu.VMEM((n, 1), jnp.float32)],
    )(x, p)


def _gru_mv_body(wih_ref, whh_ref, z_ref, h_ref, bih_ref, bhh_ref,
                 gi_ref, gh_ref):
    gi_ref[...] = lax.dot_general(wih_ref[...], z_ref[...],
                                  (((1,), (0,)), ((), ())),
                                  precision=_DEF) + bih_ref[...]
    gh_ref[...] = lax.dot_general(whh_ref[...], h_ref[...],
                                  (((1,), (0,)), ((), ())),
                                  precision=_DEF) + bhh_ref[...]


def _gru_matvec(wih, whh, z, h, bih, bhh, blk):
    rows = wih.shape[0]
    grid = rows // blk
    return pl.pallas_call(
        _gru_mv_body,
        grid=(grid,),
        in_specs=[
            pl.BlockSpec((blk, wih.shape[1]), lambda i: (i, 0)),
            pl.BlockSpec((blk, whh.shape[1]), lambda i: (i, 0)),
            pl.BlockSpec((z.shape[0], 1), lambda i: (0, 0)),
            pl.BlockSpec((h.shape[0], 1), lambda i: (0, 0)),
            pl.BlockSpec((blk, 1), lambda i: (i, 0)),
            pl.BlockSpec((blk, 1), lambda i: (i, 0)),
        ],
        out_specs=[
            pl.BlockSpec((blk, 1), lambda i: (i, 0)),
            pl.BlockSpec((blk, 1), lambda i: (i, 0)),
        ],
        out_shape=[
            jax.ShapeDtypeStruct((rows, 1), jnp.float32),
            jax.ShapeDtypeStruct((rows, 1), jnp.float32),
        ],
    )(wih, whh, z, h, bih, bhh)


def _gates_body(gi_ref, gh_ref, h_ref, out_ref):
    hsz = h_ref.shape[0]
    gi_r = gi_ref[pl.ds(0, hsz), :]
    gh_r = gh_ref[pl.ds(0, hsz), :]
    gi_z = gi_ref[pl.ds(hsz, hsz), :]
    gh_z = gh_ref[pl.ds(hsz, hsz), :]
    gi_n = gi_ref[pl.ds(2 * hsz, hsz), :]
    gh_n = gh_ref[pl.ds(2 * hsz, hsz), :]
    r = jax.nn.sigmoid(gi_r + gh_r)
    zg = jax.nn.sigmoid(gi_z + gh_z)
    ng = jnp.tanh(gi_n + r * gh_n)
    out_ref[...] = (1.0 - zg) * ng + zg * h_ref[...]


def _gates(gi, gh, h):
    return pl.pallas_call(
        _gates_body,
        out_shape=jax.ShapeDtypeStruct(h.shape, jnp.float32),
    )(gi, gh, h)


def _prep_body(h_ref, w_ref, c0_ref, c1_ref, g_ref, dinv_ref):
    deg = jnp.maximum(c0_ref[...] + c1_ref[...] + 1.0, 1.0)
    dinv = 1.0 / jnp.sqrt(deg)
    hw = lax.dot_general(h_ref[...], w_ref[...], (((1,), (0,)), ((), ())),
                         precision=_DEF)
    g_ref[...] = hw * dinv
    dinv_ref[...] = dinv


def _prep(h, w, c0, c1, blk=1000):
    n = h.shape[0]
    grid = n // blk
    return pl.pallas_call(
        _prep_body,
        grid=(grid,),
        in_specs=[
            pl.BlockSpec((blk, h.shape[1]), lambda i: (i, 0)),
            pl.BlockSpec(w.shape, lambda i: (0, 0)),
            pl.BlockSpec((blk, 1), lambda i: (i, 0)),
            pl.BlockSpec((blk, 1), lambda i: (i, 0)),
        ],
        out_specs=[
            pl.BlockSpec((blk, _DH), lambda i: (i, 0)),
            pl.BlockSpec((blk, 1), lambda i: (i, 0)),
        ],
        out_shape=[
            jax.ShapeDtypeStruct((n, _DH), jnp.float32),
            jax.ShapeDtypeStruct((n, 1), jnp.float32),
        ],
    )(h, w, c0, c1)


def _finish0_body(part_ref, g_ref, dinv_ref, b_ref, p_ref,
                  h1_ref, z_ref, y_ref):
    acc = part_ref[0] + part_ref[1] + g_ref[...]
    h1 = dinv_ref[...] * acc + b_ref[...]
    h1 = jnp.maximum(h1, 0.0)
    h1_ref[...] = h1
    p = p_ref[...]
    pn = p / (jnp.sqrt(jnp.sum(p * p)) + 1e-8)
    hb = h1.astype(jnp.bfloat16).astype(jnp.float32)
    pb = pn.astype(jnp.bfloat16).astype(jnp.float32)
    y_ref[...] = jnp.sum(hb * pb, axis=1, keepdims=True)
    n = h1_ref.shape[0]
    ids = lax.broadcasted_iota(jnp.int32, (n, 1), 0)
    kiota = lax.broadcasted_iota(jnp.int32, z_ref.shape, 0)

    def body(k, _):
        yv = y_ref[...]
        m = jnp.max(yv)
        idx = jnp.min(jnp.where(yv == m, ids, jnp.int32(n)))
        onehot = jnp.where(ids == idx, 1.0, 0.0)
        row = lax.dot_general(onehot, h1_ref[...], (((0,), (0,)), ((), ())),
                              precision=_HI)
        t = jnp.tanh(jnp.full((1, 1), m, jnp.float32))
        z_ref[...] = jnp.where(kiota == k, row * t, z_ref[...])
        y_ref[...] = jnp.where(ids == idx, -jnp.inf, yv)
        return 0

    lax.fori_loop(0, _K, body, 0)


def _finish0(part, g, dinv, b, p):
    return pl.pallas_call(
        _finish0_body,
        out_shape=[
            jax.ShapeDtypeStruct((_N, _DH), jnp.float32),
            jax.ShapeDtypeStruct((_K, _DH), jnp.float32),
        ],
        scratch_shapes=[pltpu.VMEM((_N, 1), jnp.float32)],
    )(part, g, dinv, b, p)


def _gru1_body(wih_ref, whh_ref, z_ref, h_ref, bih_ref, bhh_ref, out_ref):
    gi = lax.dot_general(wih_ref[...], z_ref[...], (((1,), (0,)), ((), ())),
                         precision=_DEF) + bih_ref[...]
    gh = lax.dot_general(whh_ref[...], h_ref[...], (((1,), (0,)), ((), ())),
                         precision=_DEF) + bhh_ref[...]
    hsz = h_ref.shape[0]
    r = jax.nn.sigmoid(gi[:hsz, :] + gh[:hsz, :])
    zg = jax.nn.sigmoid(gi[hsz:2 * hsz, :] + gh[hsz:2 * hsz, :])
    ng = jnp.tanh(gi[2 * hsz:, :] + r * gh[2 * hsz:, :])
    out_ref[...] = (1.0 - zg) * ng + zg * h_ref[...]


def _gru1(wih, whh, z, h, bih, bhh):
    return pl.pallas_call(
        _gru1_body,
        out_shape=jax.ShapeDtypeStruct(h.shape, jnp.float32),
    )(wih, whh, z, h, bih, bhh)


def _finish1_body(part_ref, g_ref, dinv_ref, b_ref, out_ref):
    acc = part_ref[0] + part_ref[1] + g_ref[...]
    out_ref[...] = dinv_ref[...] * acc + b_ref[...]


def _finish1(part, g, dinv, b, blk=1000):
    grid = _N // blk
    return pl.pallas_call(
        _finish1_body,
        grid=(grid,),
        in_specs=[
            pl.BlockSpec((2, blk, _DH), lambda i: (0, i, 0)),
            pl.BlockSpec((blk, _DH), lambda i: (i, 0)),
            pl.BlockSpec((blk, 1), lambda i: (i, 0)),
            pl.BlockSpec((1, _DH), lambda i: (0, 0)),
        ],
        out_specs=pl.BlockSpec((blk, _DH), lambda i: (i, 0)),
        out_shape=jax.ShapeDtypeStruct((_N, _DH), jnp.float32),
    )(part, g, dinv, b)


@functools.cache
def _sc_mesh():
    return plsc.VectorSubcoreMesh(core_axis_name="c", subcore_axis_name="s",
                                  num_cores=_NC, num_subcores=_NS)


def _deg_body(dst_hbm, zeros_hbm, out_hbm, idx_v, idx_t, ones_v, ones_t,
              cnt_sh, sem):
    cid = lax.axis_index("c")
    sid = lax.axis_index("s")
    base = (cid * _NS + sid) * _EPT
    for i in range(_CH // 16):
        ones_v[pl.ds(i * 16, 16)] = jnp.ones((16,), jnp.float32)
    ones_t[...] = jnp.ones((_TAIL,), jnp.float32)
    pltpu.sync_copy(zeros_hbm.at[pl.ds(sid * _RPT, _RPT)],
                    cnt_sh.at[pl.ds(sid * _RPT, _RPT)])
    plsc.subcore_barrier()

    def chunk(i, carry):
        pltpu.sync_copy(dst_hbm.at[pl.ds(base + i * _CH, _CH)], idx_v)
        pltpu.sync_copy(ones_v, cnt_sh.at[idx_v], add=True)
        return carry

    lax.fori_loop(0, _NFULL, chunk, 0)
    pltpu.sync_copy(dst_hbm.at[pl.ds(base + _NFULL * _CH, _TAIL)], idx_t)
    pltpu.sync_copy(ones_t, cnt_sh.at[idx_t], add=True)
    plsc.subcore_barrier()
    pltpu.sync_copy(cnt_sh.at[pl.ds(sid * _RPT, _RPT)],
                    out_hbm.at[cid, pl.ds(sid * _RPT, _RPT)])


@functools.cache
def _deg_kernel():
    return pl.kernel(
        _deg_body,
        out_type=jax.ShapeDtypeStruct((_NC, _NPAD), jnp.float32),
        mesh=_sc_mesh(),
        scratch_types=[
            pltpu.VMEM((_CH,), jnp.int32),
            pltpu.VMEM((_TAIL,), jnp.int32),
            pltpu.VMEM((_CH,), jnp.float32),
            pltpu.VMEM((_TAIL,), jnp.float32),
            pltpu.VMEM_SHARED((_NPAD,), jnp.float32),
            pltpu.SemaphoreType.DMA,
        ],
    )


def _deg_call(dst, zeros1):
    return _deg_kernel()(dst, zeros1)


def _edges_body(g_hbm, src_hbm, dst_hbm, zeros_hbm, out_hbm,
                src_v, dst_v, src_t, dst_t, rows_v, rows_t, acc_sh, sem):
    cid = lax.axis_index("c")
    sid = lax.axis_index("s")
    base = (cid * _NS + sid) * _EPT
    pltpu.sync_copy(zeros_hbm.at[pl.ds(sid * _RPT, _RPT)],
                    acc_sh.at[pl.ds(sid * _RPT, _RPT)])
    plsc.subcore_barrier()

    def chunk(i, carry):
        pltpu.sync_copy(src_hbm.at[pl.ds(base + i * _CH, _CH)], src_v)
        pltpu.sync_copy(dst_hbm.at[pl.ds(base + i * _CH, _CH)], dst_v)
        pltpu.async_copy(g_hbm.at[src_v], rows_v, sem).wait()
        pltpu.sync_copy(rows_v, acc_sh.at[dst_v], add=True)
        return carry

    lax.fori_loop(0, _NFULL, chunk, 0)
    pltpu.sync_copy(src_hbm.at[pl.ds(base + _NFULL * _CH, _TAIL)], src_t)
    pltpu.sync_copy(dst_hbm.at[pl.ds(base + _NFULL * _CH, _TAIL)], dst_t)
    pltpu.async_copy(g_hbm.at[src_t], rows_t, sem).wait()
    pltpu.sync_copy(rows_t, acc_sh.at[dst_t], add=True)
    plsc.subcore_barrier()
    pltpu.sync_copy(acc_sh.at[pl.ds(sid * _RPT, _RPT)],
                    out_hbm.at[cid, pl.ds(sid * _RPT, _RPT)])


@functools.cache
def _edges_kernel():
    return pl.kernel(
        _edges_body,
        out_type=jax.ShapeDtypeStruct((_NC, _NPAD, _DH), jnp.float32),
        mesh=_sc_mesh(),
        compiler_params=pltpu.CompilerParams(use_tc_tiling_on_sc=False),
        scratch_types=[
            pltpu.VMEM((_CH,), jnp.int32),
            pltpu.VMEM((_CH,), jnp.int32),
            pltpu.VMEM((_TAIL,), jnp.int32),
            pltpu.VMEM((_TAIL,), jnp.int32),
            pltpu.VMEM((_CH, _DH), jnp.float32),
            pltpu.VMEM((_TAIL, _DH), jnp.float32),
            pltpu.VMEM_SHARED((_NPAD, _DH), jnp.float32),
            pltpu.SemaphoreType.DMA,
        ],
    )


def _edges_call(g, src, dst, zeros2):
    return _edges_kernel()(g, src, dst, zeros2)


def kernel(x, edge_index, p0, p1, gcn0_h, gcn1_h, w_ih0, w_hh0, b_ih0,
           b_hh0, w_ih1, w_hh1, b_ih1, b_hh1):
    src = edge_index[0]
    dst = edge_index[1]
    zeros1 = jnp.zeros((_NPAD,), jnp.float32)
    zeros2 = jnp.zeros((_NPAD, _DH), jnp.float32)

    cnt = _deg_call(dst, zeros1)
    c0 = cnt[0, :_N].reshape(_N, 1)
    c1 = cnt[1, :_N].reshape(_N, 1)

    z0 = _summarize(x, p0.reshape(1, _D))
    z0f = z0.T.reshape(_I0, 1)
    gi0, gh0 = _gru_matvec(w_ih0, w_hh0, z0f, gcn0_h.reshape(-1, 1),
                           b_ih0.reshape(-1, 1), b_hh0.reshape(-1, 1),
                           blk=344)
    newh0 = _gates(gi0, gh0, gcn0_h.reshape(-1, 1))
    w0 = newh0[:_D * _DH].reshape(_D, _DH)
    b0 = newh0[_D * _DH:].reshape(1, _DH)
    g0, dinv = _prep(x, w0, c0, c1)
    part0 = _edges_call(g0, src, dst, zeros2)[:, :_N]
    h1, z1 = _finish0(part0, g0, dinv, b0, p1.reshape(1, _DH))

    z1f = z1.T.reshape(_I1, 1)
    newh1 = _gru1(w_ih1, w_hh1, z1f, gcn1_h.reshape(-1, 1),
                  b_ih1.reshape(-1, 1), b_hh1.reshape(-1, 1))
    w1 = newh1[:_DH * _DH].reshape(_DH, _DH)
    b1 = newh1[_DH * _DH:].reshape(1, _DH)
    g1, _ = _prep(h1, w1, c0, c1)
    part1 = _edges_call(g1, src, dst, zeros2)[:, :_N]
    out = _finish1(part1, g1, dinv, b1)
    return out


# speedup vs baseline: 15.2258x; 1.3408x over previous
import functools

import jax
import jax.numpy as jnp
from jax import lax
from jax.experimental import pallas as pl
from jax.experimental.pallas import tpu as pltpu
from jax.experimental.pallas import tpu_sc as plsc

_N = 10000
_E = 320000
_D = 128
_DH = 32
_K = 8
_H0 = _D * _DH + _DH
_I0 = _K * _D
_H1 = _DH * _DH + _DH
_I1 = _K * _DH

_NC = 2
_NS = 16
_NTILE = _NC * _NS
_EPT = _E // _NTILE
_CH = 128
_NFULL = _EPT // _CH
_TAIL = _EPT - _NFULL * _CH
_NPAD = 10240
_RPT = _NPAD // _NS

_HI = jax.lax.Precision.HIGHEST
_DEF = jax.lax.Precision.DEFAULT


def _summarize_body(x_ref, p_ref, z_ref, y_ref):
    p = p_ref[...]
    pn = p / (jnp.sqrt(jnp.sum(p * p)) + 1e-8)
    xb = x_ref[...].astype(jnp.bfloat16).astype(jnp.float32)
    pb = pn.astype(jnp.bfloat16).astype(jnp.float32)
    y_ref[...] = jnp.sum(xb * pb, axis=1, keepdims=True)
    n = x_ref.shape[0]
    ids = lax.broadcasted_iota(jnp.int32, (n, 1), 0)
    kiota = lax.broadcasted_iota(jnp.int32, z_ref.shape, 0)

    def body(k, _):
        yv = y_ref[...]
        m = jnp.max(yv)
        idx = jnp.min(jnp.where(yv == m, ids, jnp.int32(n)))
        onehot = jnp.where(ids == idx, 1.0, 0.0)
        row = lax.dot_general(onehot, x_ref[...], (((0,), (0,)), ((), ())),
                              precision=_HI)
        t = jnp.tanh(jnp.full((1, 1), m, jnp.float32))
        z_ref[...] = jnp.where(kiota == k, row * t, z_ref[...])
        y_ref[...] = jnp.where(ids == idx, -jnp.inf, yv)
        return 0

    lax.fori_loop(0, _K, body, 0)


def _summarize(x, p):
    n, d = x.shape
    return pl.pallas_call(
        _summarize_body,
        out_shape=jax.ShapeDtypeStruct((_K, d), jnp.float32),
        scratch_shapes=[pltpu.VMEM((n, 1), jnp.float32)],
    )(x, p)


def _gru_mv_body(wih_ref, whh_ref, z_ref, h_ref, bih_ref, bhh_ref,
                 gi_ref, gh_ref):
    gi_ref[...] = lax.dot_general(wih_ref[...], z_ref[...],
                                  (((1,), (0,)), ((), ())),
                                  precision=_DEF) + bih_ref[...]
    gh_ref[...] = lax.dot_general(whh_ref[...], h_ref[...],
                                  (((1,), (0,)), ((), ())),
                                  precision=_DEF) + bhh_ref[...]


def _gru_matvec(wih, whh, z, h, bih, bhh, blk):
    rows = wih.shape[0]
    grid = rows // blk
    return pl.pallas_call(
        _gru_mv_body,
        grid=(grid,),
        in_specs=[
            pl.BlockSpec((blk, wih.shape[1]), lambda i: (i, 0)),
            pl.BlockSpec((blk, whh.shape[1]), lambda i: (i, 0)),
            pl.BlockSpec((z.shape[0], 1), lambda i: (0, 0)),
            pl.BlockSpec((h.shape[0], 1), lambda i: (0, 0)),
            pl.BlockSpec((blk, 1), lambda i: (i, 0)),
            pl.BlockSpec((blk, 1), lambda i: (i, 0)),
        ],
        out_specs=[
            pl.BlockSpec((blk, 1), lambda i: (i, 0)),
            pl.BlockSpec((blk, 1), lambda i: (i, 0)),
        ],
        out_shape=[
            jax.ShapeDtypeStruct((rows, 1), jnp.float32),
            jax.ShapeDtypeStruct((rows, 1), jnp.float32),
        ],
    )(wih, whh, z, h, bih, bhh)


def _gates_body(gi_ref, gh_ref, h_ref, out_ref):
    hsz = h_ref.shape[0]
    gi_r = gi_ref[pl.ds(0, hsz), :]
    gh_r = gh_ref[pl.ds(0, hsz), :]
    gi_z = gi_ref[pl.ds(hsz, hsz), :]
    gh_z = gh_ref[pl.ds(hsz, hsz), :]
    gi_n = gi_ref[pl.ds(2 * hsz, hsz), :]
    gh_n = gh_ref[pl.ds(2 * hsz, hsz), :]
    r = jax.nn.sigmoid(gi_r + gh_r)
    zg = jax.nn.sigmoid(gi_z + gh_z)
    ng = jnp.tanh(gi_n + r * gh_n)
    out_ref[...] = (1.0 - zg) * ng + zg * h_ref[...]


def _gates(gi, gh, h):
    return pl.pallas_call(
        _gates_body,
        out_shape=jax.ShapeDtypeStruct(h.shape, jnp.float32),
    )(gi, gh, h)


def _prep_body(h_ref, w_ref, c0_ref, c1_ref, g_ref, dinv_ref):
    deg = jnp.maximum(c0_ref[...] + c1_ref[...] + 1.0, 1.0)
    dinv = 1.0 / jnp.sqrt(deg)
    hw = lax.dot_general(h_ref[...], w_ref[...], (((1,), (0,)), ((), ())),
                         precision=_DEF)
    g_ref[...] = hw * dinv
    dinv_ref[...] = dinv


def _prep(h, w, c0, c1, blk=1000):
    n = h.shape[0]
    grid = n // blk
    return pl.pallas_call(
        _prep_body,
        grid=(grid,),
        in_specs=[
            pl.BlockSpec((blk, h.shape[1]), lambda i: (i, 0)),
            pl.BlockSpec(w.shape, lambda i: (0, 0)),
            pl.BlockSpec((blk, 1), lambda i: (i, 0)),
            pl.BlockSpec((blk, 1), lambda i: (i, 0)),
        ],
        out_specs=[
            pl.BlockSpec((blk, _DH), lambda i: (i, 0)),
            pl.BlockSpec((blk, 1), lambda i: (i, 0)),
        ],
        out_shape=[
            jax.ShapeDtypeStruct((n, _DH), jnp.float32),
            jax.ShapeDtypeStruct((n, 1), jnp.float32),
        ],
    )(h, w, c0, c1)


def _finish0_body(part_ref, g_ref, dinv_ref, b_ref, p_ref,
                  h1_ref, z_ref, y_ref):
    acc = part_ref[0] + part_ref[1] + g_ref[...]
    h1 = dinv_ref[...] * acc + b_ref[...]
    h1 = jnp.maximum(h1, 0.0)
    h1_ref[...] = h1
    p = p_ref[...]
    pn = p / (jnp.sqrt(jnp.sum(p * p)) + 1e-8)
    hb = h1.astype(jnp.bfloat16).astype(jnp.float32)
    pb = pn.astype(jnp.bfloat16).astype(jnp.float32)
    y_ref[...] = jnp.sum(hb * pb, axis=1, keepdims=True)
    n = h1_ref.shape[0]
    ids = lax.broadcasted_iota(jnp.int32, (n, 1), 0)
    kiota = lax.broadcasted_iota(jnp.int32, z_ref.shape, 0)

    def body(k, _):
        yv = y_ref[...]
        m = jnp.max(yv)
        idx = jnp.min(jnp.where(yv == m, ids, jnp.int32(n)))
        onehot = jnp.where(ids == idx, 1.0, 0.0)
        row = lax.dot_general(onehot, h1_ref[...], (((0,), (0,)), ((), ())),
                              precision=_HI)
        t = jnp.tanh(jnp.full((1, 1), m, jnp.float32))
        z_ref[...] = jnp.where(kiota == k, row * t, z_ref[...])
        y_ref[...] = jnp.where(ids == idx, -jnp.inf, yv)
        return 0

    lax.fori_loop(0, _K, body, 0)


def _finish0(part, g, dinv, b, p):
    return pl.pallas_call(
        _finish0_body,
        out_shape=[
            jax.ShapeDtypeStruct((_N, _DH), jnp.float32),
            jax.ShapeDtypeStruct((_K, _DH), jnp.float32),
        ],
        scratch_shapes=[pltpu.VMEM((_N, 1), jnp.float32)],
    )(part, g, dinv, b, p)


def _gru1_body(wih_ref, whh_ref, z_ref, h_ref, bih_ref, bhh_ref, out_ref):
    gi = lax.dot_general(wih_ref[...], z_ref[...], (((1,), (0,)), ((), ())),
                         precision=_DEF) + bih_ref[...]
    gh = lax.dot_general(whh_ref[...], h_ref[...], (((1,), (0,)), ((), ())),
                         precision=_DEF) + bhh_ref[...]
    hsz = h_ref.shape[0]
    r = jax.nn.sigmoid(gi[:hsz, :] + gh[:hsz, :])
    zg = jax.nn.sigmoid(gi[hsz:2 * hsz, :] + gh[hsz:2 * hsz, :])
    ng = jnp.tanh(gi[2 * hsz:, :] + r * gh[2 * hsz:, :])
    out_ref[...] = (1.0 - zg) * ng + zg * h_ref[...]


def _gru1(wih, whh, z, h, bih, bhh):
    return pl.pallas_call(
        _gru1_body,
        out_shape=jax.ShapeDtypeStruct(h.shape, jnp.float32),
    )(wih, whh, z, h, bih, bhh)


def _finish1_body(part_ref, g_ref, dinv_ref, b_ref, out_ref):
    acc = part_ref[0] + part_ref[1] + g_ref[...]
    out_ref[...] = dinv_ref[...] * acc + b_ref[...]


def _finish1(part, g, dinv, b, blk=1000):
    grid = _N // blk
    return pl.pallas_call(
        _finish1_body,
        grid=(grid,),
        in_specs=[
            pl.BlockSpec((2, blk, _DH), lambda i: (0, i, 0)),
            pl.BlockSpec((blk, _DH), lambda i: (i, 0)),
            pl.BlockSpec((blk, 1), lambda i: (i, 0)),
            pl.BlockSpec((1, _DH), lambda i: (0, 0)),
        ],
        out_specs=pl.BlockSpec((blk, _DH), lambda i: (i, 0)),
        out_shape=jax.ShapeDtypeStruct((_N, _DH), jnp.float32),
    )(part, g, dinv, b)


@functools.cache
def _sc_mesh():
    return plsc.VectorSubcoreMesh(core_axis_name="c", subcore_axis_name="s",
                                  num_cores=_NC, num_subcores=_NS)


def _deg_body(dst_hbm, zeros_hbm, out_hbm, idx_v, idx_t, ones_v, ones_t,
              cnt_sh, sem):
    cid = lax.axis_index("c")
    sid = lax.axis_index("s")
    base = (cid * _NS + sid) * _EPT
    for i in range(_CH // 16):
        ones_v[pl.ds(i * 16, 16)] = jnp.ones((16,), jnp.float32)
    ones_t[...] = jnp.ones((_TAIL,), jnp.float32)
    pltpu.sync_copy(zeros_hbm.at[pl.ds(sid * _RPT, _RPT)],
                    cnt_sh.at[pl.ds(sid * _RPT, _RPT)])
    plsc.subcore_barrier()

    def chunk(i, carry):
        pltpu.sync_copy(dst_hbm.at[pl.ds(base + i * _CH, _CH)], idx_v)
        pltpu.sync_copy(ones_v, cnt_sh.at[idx_v], add=True)
        return carry

    lax.fori_loop(0, _NFULL, chunk, 0)
    pltpu.sync_copy(dst_hbm.at[pl.ds(base + _NFULL * _CH, _TAIL)], idx_t)
    pltpu.sync_copy(ones_t, cnt_sh.at[idx_t], add=True)
    plsc.subcore_barrier()
    pltpu.sync_copy(cnt_sh.at[pl.ds(sid * _RPT, _RPT)],
                    out_hbm.at[cid, pl.ds(sid * _RPT, _RPT)])


@functools.cache
def _deg_kernel():
    return pl.kernel(
        _deg_body,
        out_type=jax.ShapeDtypeStruct((_NC, _NPAD), jnp.float32),
        mesh=_sc_mesh(),
        scratch_types=[
            pltpu.VMEM((_CH,), jnp.int32),
            pltpu.VMEM((_TAIL,), jnp.int32),
            pltpu.VMEM((_CH,), jnp.float32),
            pltpu.VMEM((_TAIL,), jnp.float32),
            pltpu.VMEM_SHARED((_NPAD,), jnp.float32),
            pltpu.SemaphoreType.DMA,
        ],
    )


def _deg_call(dst, zeros1):
    return _deg_kernel()(dst, zeros1)


def _edges_body(g_hbm, src_hbm, dst_hbm, zeros_hbm, out_hbm,
                srcb, dstb, rows, acc_sh, gsem0, gsem1, sem):
    cid = lax.axis_index("c")
    sid = lax.axis_index("s")
    base = (cid * _NS + sid) * _EPT
    pltpu.sync_copy(zeros_hbm.at[pl.ds(sid * _RPT, _RPT)],
                    acc_sh.at[pl.ds(sid * _RPT, _RPT)])
    pltpu.sync_copy(src_hbm.at[pl.ds(base, _EPT)], srcb)
    pltpu.sync_copy(dst_hbm.at[pl.ds(base, _EPT)], dstb)
    plsc.subcore_barrier()

    pltpu.async_copy(g_hbm.at[srcb.at[pl.ds(0, _CH)]], rows.at[0], gsem0)

    def chunk(i, carry):
        nxt = (i + 1) * _CH

        @pl.when(jnp.logical_and(i + 1 < _NFULL, (i & 1) == 0))
        def _():
            pltpu.async_copy(g_hbm.at[srcb.at[pl.ds(nxt, _CH)]],
                             rows.at[1], gsem1)

        @pl.when(jnp.logical_and(i + 1 < _NFULL, (i & 1) == 1))
        def _():
            pltpu.async_copy(g_hbm.at[srcb.at[pl.ds(nxt, _CH)]],
                             rows.at[0], gsem0)

        @pl.when((i & 1) == 0)
        def _():
            pltpu.make_async_copy(g_hbm.at[srcb.at[pl.ds(i * _CH, _CH)]],
                                  rows.at[0], gsem0).wait()
            pltpu.sync_copy(rows.at[0],
                            acc_sh.at[dstb.at[pl.ds(i * _CH, _CH)]], add=True)

        @pl.when((i & 1) == 1)
        def _():
            pltpu.make_async_copy(g_hbm.at[srcb.at[pl.ds(i * _CH, _CH)]],
                                  rows.at[1], gsem1).wait()
            pltpu.sync_copy(rows.at[1],
                            acc_sh.at[dstb.at[pl.ds(i * _CH, _CH)]], add=True)

        return carry

    lax.fori_loop(0, _NFULL, chunk, 0)
    pltpu.async_copy(g_hbm.at[srcb.at[pl.ds(_NFULL * _CH, _TAIL)]],
                     rows.at[0, pl.ds(0, _TAIL)], gsem0).wait()
    pltpu.sync_copy(rows.at[0, pl.ds(0, _TAIL)],
                    acc_sh.at[dstb.at[pl.ds(_NFULL * _CH, _TAIL)]], add=True)
    plsc.subcore_barrier()
    pltpu.sync_copy(acc_sh.at[pl.ds(sid * _RPT, _RPT)],
                    out_hbm.at[cid, pl.ds(sid * _RPT, _RPT)])


@functools.cache
def _edges_kernel():
    return pl.kernel(
        _edges_body,
        out_type=jax.ShapeDtypeStruct((_NC, _NPAD, _DH), jnp.float32),
        mesh=_sc_mesh(),
        compiler_params=pltpu.CompilerParams(use_tc_tiling_on_sc=False),
        scratch_types=[
            pltpu.VMEM((_EPT,), jnp.int32),
            pltpu.VMEM((_EPT,), jnp.int32),
            pltpu.VMEM((2, _CH, _DH), jnp.float32),
            pltpu.VMEM_SHARED((_NPAD, _DH), jnp.float32),
            pltpu.SemaphoreType.DMA,
            pltpu.SemaphoreType.DMA,
            pltpu.SemaphoreType.DMA,
        ],
    )


def _edges_call(g, src, dst, zeros2):
    return _edges_kernel()(g, src, dst, zeros2)


def kernel(x, edge_index, p0, p1, gcn0_h, gcn1_h, w_ih0, w_hh0, b_ih0,
           b_hh0, w_ih1, w_hh1, b_ih1, b_hh1):
    src = edge_index[0]
    dst = edge_index[1]
    zeros1 = jnp.zeros((_NPAD,), jnp.float32)
    zeros2 = jnp.zeros((_NPAD, _DH), jnp.float32)

    cnt = _deg_call(dst, zeros1)
    c0 = cnt[0, :_N].reshape(_N, 1)
    c1 = cnt[1, :_N].reshape(_N, 1)

    z0 = _summarize(x, p0.reshape(1, _D))
    z0f = z0.T.reshape(_I0, 1)
    gi0, gh0 = _gru_matvec(w_ih0, w_hh0, z0f, gcn0_h.reshape(-1, 1),
                           b_ih0.reshape(-1, 1), b_hh0.reshape(-1, 1),
                           blk=344)
    newh0 = _gates(gi0, gh0, gcn0_h.reshape(-1, 1))
    w0 = newh0[:_D * _DH].reshape(_D, _DH)
    b0 = newh0[_D * _DH:].reshape(1, _DH)
    g0, dinv = _prep(x, w0, c0, c1)
    part0 = _edges_call(g0, src, dst, zeros2)[:, :_N]
    h1, z1 = _finish0(part0, g0, dinv, b0, p1.reshape(1, _DH))

    z1f = z1.T.reshape(_I1, 1)
    newh1 = _gru1(w_ih1, w_hh1, z1f, gcn1_h.reshape(-1, 1),
                  b_ih1.reshape(-1, 1), b_hh1.reshape(-1, 1))
    w1 = newh1[:_DH * _DH].reshape(_DH, _DH)
    b1 = newh1[_DH * _DH:].reshape(1, _DH)
    g1, _ = _prep(h1, w1, c0, c1)
    part1 = _edges_call(g1, src, dst, zeros2)[:, :_N]
    out = _finish1(part1, g1, dinv, b1)
    return out
